# baseline (speedup 1.0000x reference)
import jax
import jax.numpy as jnp
from jax import lax
from jax.experimental import pallas as pl
from jax.experimental.pallas import tpu as pltpu

B, N, K, DIM, NBLK = 4, 2500, 8, 128, 4
NP = 2560
T = 256
NT = NP // T
CW = 512
NC = NP // CW
BIG = 1e30
ISENT = NP


def _net_kernel(h0_ref, W1_ref, b1_ref, W2_ref, b2_ref,
                Wp1_ref, bp1_ref, lng_ref, lnb_ref, Wp2_ref, bp2_ref,
                h_ref, off_ref, h1_s, hw_s, idx_s, sq_s):
    h_ref[0] = h0_ref[0]

    lio = lax.broadcasted_iota(jnp.int32, (T, CW), 1)
    rowio = lax.broadcasted_iota(jnp.int32, (T, 1), 0)
    k8 = lax.broadcasted_iota(jnp.int32, (1, K), 1)

    def agg_pass(i8):
        def agg_chunk(c, agg):
            c0 = c * CW
            acc = jnp.zeros((T, CW), jnp.float32)
            lcg = c0 + lio
            for k in range(K):
                acc = acc + (lcg == i8[:, k:k + 1]).astype(jnp.float32)
            hwc = hw_s[pl.ds(c0, CW), :]
            return agg + jnp.dot(acc, hwc, precision=lax.Precision.HIGHEST)

        return lax.fori_loop(0, NC, agg_chunk,
                             jnp.zeros((T, DIM), jnp.float32))

    def block_body(blk, _):
        W1i = W1_ref[blk]
        W2i = W2_ref[blk]
        b1i = b1_ref[pl.ds(blk, 1), :]
        b2i = b2_ref[pl.ds(blk, 1), :]
        hb = h_ref[0]
        sq_s[:] = jnp.sum(hb * hb, axis=1)[None, :]
        hw_s[:] = jnp.dot(hb, W1i) * (1.0 / 9.0)

        def p1_tile(t, _):
            r0 = t * T
            ht = h_ref[0, pl.ds(r0, T), :]

            def chunk_body(c, carry):
                v8, i8 = carry
                c0 = c * CW
                hbc = h_ref[0, pl.ds(c0, CW), :]
                m2 = lax.dot_general(ht, hbc, (((1,), (1,)), ((), ())))
                dc = sq_s[:, pl.ds(c0, CW)] - (m2 + m2)
                lcg = c0 + lio
                dc = lax.cond(
                    c0 == (r0 // CW) * CW,
                    lambda d: jnp.where(lcg == r0 + rowio, d + 1e10, d),
                    lambda d: d, dc)
                dc = lax.cond(
                    c == NC - 1,
                    lambda d: jnp.where(lcg >= N, BIG, d),
                    lambda d: d, dc)

                def sel(k, cr):
                    dc, v8, i8, nv, ni = cr
                    mnp = jnp.minimum(
                        jnp.min(dc, axis=1, keepdims=True),
                        jnp.min(v8, axis=1, keepdims=True))
                    cd = jnp.min(jnp.where(dc == mnp, lcg, ISENT),
                                 axis=1, keepdims=True)
                    cv = jnp.min(jnp.where(v8 == mnp, i8, ISENT),
                                 axis=1, keepdims=True)
                    a = jnp.minimum(cd, cv)
                    dc = jnp.where(lcg == a, BIG, dc)
                    v8 = jnp.where(i8 == a, BIG, v8)
                    nv = jnp.where(k8 == k, mnp, nv)
                    ni = jnp.where(k8 == k, a, ni)
                    return dc, v8, i8, nv, ni

                nv0 = jnp.full((T, K), BIG, jnp.float32)
                ni0 = jnp.full((T, K), ISENT, jnp.int32)
                _, _, _, nv, ni = lax.fori_loop(0, K, sel,
                                                (dc, v8, i8, nv0, ni0))
                return nv, ni

            v80 = jnp.full((T, K), BIG, jnp.float32)
            i80 = jnp.full((T, K), ISENT, jnp.int32)
            _, i8 = lax.fori_loop(0, NC, chunk_body, (v80, i80))
            idx_s[pl.ds(r0, T), :] = i8
            agg = agg_pass(i8)
            h1t = jax.nn.relu(agg + hw_s[pl.ds(r0, T), :] + b1i)
            h1_s[pl.ds(r0, T), :] = h1t
            return 0

        lax.fori_loop(0, NT, p1_tile, 0)
        hw_s[:] = jnp.dot(h1_s[:], W2i) * (1.0 / 9.0)

        def p2_tile(t, _):
            r0 = t * T
            idxt = idx_s[pl.ds(r0, T), :]
            agg = agg_pass(idxt)
            out = (agg + hw_s[pl.ds(r0, T), :] + b2i
                   + h_ref[0, pl.ds(r0, T), :])
            h_ref[0, pl.ds(r0, T), :] = out
            return 0

        lax.fori_loop(0, NT, p2_tile, 0)
        return 0

    lax.fori_loop(0, NBLK, block_body, 0)

    hf = h_ref[0]
    z = jnp.dot(hf, Wp1_ref[:]) + bp1_ref[:]
    mu = jnp.mean(z, axis=1, keepdims=True)
    zc = z - mu
    var = jnp.mean(zc * zc, axis=1, keepdims=True)
    z = zc / jnp.sqrt(var + 1e-5) * lng_ref[:] + lnb_ref[:]
    z = jax.nn.relu(z)
    g = jnp.dot(z, Wp2_ref[:]) + bp2_ref[:]
    off = jnp.tanh(g) * 0.1
    off_ref[0] = off[:, :8]


@jax.jit
def kernel(xyz, features, W1, b1, W2, b2, Wp1, bp1, ln_g, ln_b, Wp2, bp2):
    h0 = jnp.transpose(features, (0, 2, 1))
    h0 = jnp.pad(h0, ((0, 0), (0, NP - N), (0, 0)))
    Wp2p = jnp.pad(Wp2, ((0, 0), (0, 128 - Wp2.shape[1])))
    bp2p = jnp.pad(bp2, (0, 128 - bp2.shape[0]))[None, :]
    bp1r = bp1[None, :]
    lngr = ln_g[None, :]
    lnbr = ln_b[None, :]

    hout, off = pl.pallas_call(
        _net_kernel,
        grid=(B,),
        in_specs=[
            pl.BlockSpec((1, NP, DIM), lambda b: (b, 0, 0)),
            pl.BlockSpec((NBLK, DIM, DIM), lambda b: (0, 0, 0)),
            pl.BlockSpec((NBLK, DIM), lambda b: (0, 0)),
            pl.BlockSpec((NBLK, DIM, DIM), lambda b: (0, 0, 0)),
            pl.BlockSpec((NBLK, DIM), lambda b: (0, 0)),
            pl.BlockSpec((DIM, 128), lambda b: (0, 0)),
            pl.BlockSpec((1, 128), lambda b: (0, 0)),
            pl.BlockSpec((1, 128), lambda b: (0, 0)),
            pl.BlockSpec((1, 128), lambda b: (0, 0)),
            pl.BlockSpec((DIM, 128), lambda b: (0, 0)),
            pl.BlockSpec((1, 128), lambda b: (0, 0)),
        ],
        out_specs=[
            pl.BlockSpec((1, NP, DIM), lambda b: (b, 0, 0)),
            pl.BlockSpec((1, NP, 8), lambda b: (b, 0, 0)),
        ],
        out_shape=[
            jax.ShapeDtypeStruct((B, NP, DIM), jnp.float32),
            jax.ShapeDtypeStruct((B, NP, 8), jnp.float32),
        ],
        scratch_shapes=[
            pltpu.VMEM((NP, DIM), jnp.float32),
            pltpu.VMEM((NP, DIM), jnp.float32),
            pltpu.VMEM((NP, K), jnp.int32),
            pltpu.VMEM((1, NP), jnp.float32),
        ],
        compiler_params=pltpu.CompilerParams(
            dimension_semantics=("parallel",)),
    )(h0, W1, b1, W2, b2, Wp1, bp1r, lngr, lnbr, Wp2p, bp2p)

    h = hout[:, :N, :]
    new_features = jnp.transpose(h, (0, 2, 1))
    off6 = off[:, :N, :6].reshape(B, N, 3, 2)
    new_xyz = (xyz[:, :, None, :]
               + jnp.transpose(off6, (0, 2, 3, 1))).reshape(B, 3, 2 * N)
    return (new_xyz, new_features)


# speedup vs baseline: 1.0764x; 1.0764x over previous
import jax
import jax.numpy as jnp
from jax import lax
from jax.experimental import pallas as pl
from jax.experimental.pallas import tpu as pltpu

B, N, K, DIM, NBLK = 4, 2500, 8, 128, 4
NP = 2560
T = 256
NT = NP // T
CW = 512
NC = NP // CW
BIG = 1e30
ISENT = NP


def _net_kernel(h0_ref, W1_ref, b1_ref, W2_ref, b2_ref,
                Wp1_ref, bp1_ref, lng_ref, lnb_ref, Wp2_ref, bp2_ref,
                h_ref, off_ref, h1_s, hw_s, idx_s, sq_s, hwa_s, hwb_s, hwc_s):
    h_ref[0] = h0_ref[0]

    lio = lax.broadcasted_iota(jnp.int32, (T, CW), 1)
    rowio = lax.broadcasted_iota(jnp.int32, (T, 1), 0)
    k8 = lax.broadcasted_iota(jnp.int32, (1, K), 1)

    def set_hw(hw):
        hw_s[:] = hw
        a = hw.astype(jnp.bfloat16)
        r = hw - a.astype(jnp.float32)
        b = r.astype(jnp.bfloat16)
        c = (r - b.astype(jnp.float32)).astype(jnp.bfloat16)
        hwa_s[:] = a
        hwb_s[:] = b
        hwc_s[:] = c

    def agg_pass(i8):
        def agg_chunk(c, agg):
            c0 = c * CW
            acc = jnp.zeros((T, CW), jnp.bfloat16)
            lcg = c0 + lio
            for k in range(K):
                acc = acc + (lcg == i8[:, k:k + 1]).astype(jnp.bfloat16)
            p = jnp.dot(acc, hwa_s[pl.ds(c0, CW), :],
                        preferred_element_type=jnp.float32)
            p = p + jnp.dot(acc, hwb_s[pl.ds(c0, CW), :],
                            preferred_element_type=jnp.float32)
            p = p + jnp.dot(acc, hwc_s[pl.ds(c0, CW), :],
                            preferred_element_type=jnp.float32)
            return agg + p

        return lax.fori_loop(0, NC, agg_chunk,
                             jnp.zeros((T, DIM), jnp.float32))

    def block_body(blk, _):
        W1i = W1_ref[blk]
        W2i = W2_ref[blk]
        b1i = b1_ref[pl.ds(blk, 1), :]
        b2i = b2_ref[pl.ds(blk, 1), :]
        hb = h_ref[0]
        sq_s[:] = jnp.sum(hb * hb, axis=1)[None, :]
        set_hw(jnp.dot(hb, W1i) * (1.0 / 9.0))

        def p1_tile(t, _):
            r0 = t * T
            ht = h_ref[0, pl.ds(r0, T), :]

            def chunk_body(c, carry):
                v8, i8 = carry
                c0 = c * CW
                hbc = h_ref[0, pl.ds(c0, CW), :]
                m2 = lax.dot_general(ht, hbc, (((1,), (1,)), ((), ())))
                dc = sq_s[:, pl.ds(c0, CW)] - (m2 + m2)
                lcg = c0 + lio
                dc = lax.cond(
                    c0 == (r0 // CW) * CW,
                    lambda d: jnp.where(lcg == r0 + rowio, d + 1e10, d),
                    lambda d: d, dc)
                dc = lax.cond(
                    c == NC - 1,
                    lambda d: jnp.where(lcg >= N, BIG, d),
                    lambda d: d, dc)

                def sel(k, cr):
                    dc, v8, i8, nv, ni = cr
                    mnp = jnp.minimum(
                        jnp.min(dc, axis=1, keepdims=True),
                        jnp.min(v8, axis=1, keepdims=True))
                    cd = jnp.min(jnp.where(dc == mnp, lcg, ISENT),
                                 axis=1, keepdims=True)
                    cv = jnp.min(jnp.where(v8 == mnp, i8, ISENT),
                                 axis=1, keepdims=True)
                    a = jnp.minimum(cd, cv)
                    dc = jnp.where(lcg == a, BIG, dc)
                    v8 = jnp.where(i8 == a, BIG, v8)
                    nv = jnp.where(k8 == k, mnp, nv)
                    ni = jnp.where(k8 == k, a, ni)
                    return dc, v8, i8, nv, ni

                nv0 = jnp.full((T, K), BIG, jnp.float32)
                ni0 = jnp.full((T, K), ISENT, jnp.int32)
                _, _, _, nv, ni = lax.fori_loop(0, K, sel,
                                                (dc, v8, i8, nv0, ni0))
                return nv, ni

            v80 = jnp.full((T, K), BIG, jnp.float32)
            i80 = jnp.full((T, K), ISENT, jnp.int32)
            _, i8 = lax.fori_loop(0, NC, chunk_body, (v80, i80))
            idx_s[pl.ds(r0, T), :] = i8
            agg = agg_pass(i8)
            h1t = jax.nn.relu(agg + hw_s[pl.ds(r0, T), :] + b1i)
            h1_s[pl.ds(r0, T), :] = h1t
            return 0

        lax.fori_loop(0, NT, p1_tile, 0)
        set_hw(jnp.dot(h1_s[:], W2i) * (1.0 / 9.0))

        def p2_tile(t, _):
            r0 = t * T
            idxt = idx_s[pl.ds(r0, T), :]
            agg = agg_pass(idxt)
            out = (agg + hw_s[pl.ds(r0, T), :] + b2i
                   + h_ref[0, pl.ds(r0, T), :])
            h_ref[0, pl.ds(r0, T), :] = out
            return 0

        lax.fori_loop(0, NT, p2_tile, 0)
        return 0

    lax.fori_loop(0, NBLK, block_body, 0)

    hf = h_ref[0]
    z = jnp.dot(hf, Wp1_ref[:]) + bp1_ref[:]
    mu = jnp.mean(z, axis=1, keepdims=True)
    zc = z - mu
    var = jnp.mean(zc * zc, axis=1, keepdims=True)
    z = zc / jnp.sqrt(var + 1e-5) * lng_ref[:] + lnb_ref[:]
    z = jax.nn.relu(z)
    g = jnp.dot(z, Wp2_ref[:]) + bp2_ref[:]
    off = jnp.tanh(g) * 0.1
    off_ref[0] = off[:, :8]


@jax.jit
def kernel(xyz, features, W1, b1, W2, b2, Wp1, bp1, ln_g, ln_b, Wp2, bp2):
    h0 = jnp.transpose(features, (0, 2, 1))
    h0 = jnp.pad(h0, ((0, 0), (0, NP - N), (0, 0)))
    Wp2p = jnp.pad(Wp2, ((0, 0), (0, 128 - Wp2.shape[1])))
    bp2p = jnp.pad(bp2, (0, 128 - bp2.shape[0]))[None, :]
    bp1r = bp1[None, :]
    lngr = ln_g[None, :]
    lnbr = ln_b[None, :]

    hout, off = pl.pallas_call(
        _net_kernel,
        grid=(B,),
        in_specs=[
            pl.BlockSpec((1, NP, DIM), lambda b: (b, 0, 0)),
            pl.BlockSpec((NBLK, DIM, DIM), lambda b: (0, 0, 0)),
            pl.BlockSpec((NBLK, DIM), lambda b: (0, 0)),
            pl.BlockSpec((NBLK, DIM, DIM), lambda b: (0, 0, 0)),
            pl.BlockSpec((NBLK, DIM), lambda b: (0, 0)),
            pl.BlockSpec((DIM, 128), lambda b: (0, 0)),
            pl.BlockSpec((1, 128), lambda b: (0, 0)),
            pl.BlockSpec((1, 128), lambda b: (0, 0)),
            pl.BlockSpec((1, 128), lambda b: (0, 0)),
            pl.BlockSpec((DIM, 128), lambda b: (0, 0)),
            pl.BlockSpec((1, 128), lambda b: (0, 0)),
        ],
        out_specs=[
            pl.BlockSpec((1, NP, DIM), lambda b: (b, 0, 0)),
            pl.BlockSpec((1, NP, 8), lambda b: (b, 0, 0)),
        ],
        out_shape=[
            jax.ShapeDtypeStruct((B, NP, DIM), jnp.float32),
            jax.ShapeDtypeStruct((B, NP, 8), jnp.float32),
        ],
        scratch_shapes=[
            pltpu.VMEM((NP, DIM), jnp.float32),
            pltpu.VMEM((NP, DIM), jnp.float32),
            pltpu.VMEM((NP, K), jnp.int32),
            pltpu.VMEM((1, NP), jnp.float32),
            pltpu.VMEM((NP, DIM), jnp.bfloat16),
            pltpu.VMEM((NP, DIM), jnp.bfloat16),
            pltpu.VMEM((NP, DIM), jnp.bfloat16),
        ],
        compiler_params=pltpu.CompilerParams(
            dimension_semantics=("parallel",)),
    )(h0, W1, b1, W2, b2, Wp1, bp1r, lngr, lnbr, Wp2p, bp2p)

    h = hout[:, :N, :]
    new_features = jnp.transpose(h, (0, 2, 1))
    off6 = off[:, :N, :6].reshape(B, N, 3, 2)
    new_xyz = (xyz[:, :, None, :]
               + jnp.transpose(off6, (0, 2, 3, 1))).reshape(B, 3, 2 * N)
    return (new_xyz, new_features)


# speedup vs baseline: 1.2522x; 1.1633x over previous
import jax
import jax.numpy as jnp
from jax import lax
from jax.experimental import pallas as pl
from jax.experimental.pallas import tpu as pltpu

B, N, K, DIM, NBLK = 4, 2500, 8, 128, 4
NP = 2560
T = 512
NT = NP // T
CW = 512
NC = NP // CW
BIG = 1e30
ISENT = NP


def _net_kernel(h0_ref, W1_ref, b1_ref, W2_ref, b2_ref,
                Wp1_ref, bp1_ref, lng_ref, lnb_ref, Wp2_ref, bp2_ref,
                h_ref, off_ref, h1_s, hw_s, idx_s, sq_s, hwa_s, hwb_s, hwc_s):
    h_ref[0] = h0_ref[0]

    lio = lax.broadcasted_iota(jnp.int32, (T, CW), 1)
    rowio = lax.broadcasted_iota(jnp.int32, (T, 1), 0)
    k8 = lax.broadcasted_iota(jnp.int32, (1, K), 1)

    def set_hw(hw):
        hw_s[:] = hw
        a = hw.astype(jnp.bfloat16)
        r = hw - a.astype(jnp.float32)
        b = r.astype(jnp.bfloat16)
        c = (r - b.astype(jnp.float32)).astype(jnp.bfloat16)
        hwa_s[:] = a
        hwb_s[:] = b
        hwc_s[:] = c

    def agg_pass(i8):
        def agg_chunk(c, agg):
            c0 = c * CW
            acc = jnp.zeros((T, CW), jnp.bfloat16)
            lcg = c0 + lio
            for k in range(K):
                acc = acc + (lcg == i8[:, k:k + 1]).astype(jnp.bfloat16)
            p = jnp.dot(acc, hwa_s[pl.ds(c0, CW), :],
                        preferred_element_type=jnp.float32)
            p = p + jnp.dot(acc, hwb_s[pl.ds(c0, CW), :],
                            preferred_element_type=jnp.float32)
            p = p + jnp.dot(acc, hwc_s[pl.ds(c0, CW), :],
                            preferred_element_type=jnp.float32)
            return agg + p

        return lax.fori_loop(0, NC, agg_chunk,
                             jnp.zeros((T, DIM), jnp.float32))

    def block_body(blk, _):
        W1i = W1_ref[blk]
        W2i = W2_ref[blk]
        b1i = b1_ref[pl.ds(blk, 1), :]
        b2i = b2_ref[pl.ds(blk, 1), :]
        hb = h_ref[0]
        sq_s[:] = jnp.sum(hb * hb, axis=1)[None, :]
        set_hw(jnp.dot(hb, W1i) * (1.0 / 9.0))

        def p1_tile(t, _):
            r0 = t * T
            ht = h_ref[0, pl.ds(r0, T), :]

            def chunk_body(c, carry):
                v8, i8 = carry
                c0 = c * CW
                hbc = h_ref[0, pl.ds(c0, CW), :]
                m2 = lax.dot_general(ht, hbc, (((1,), (1,)), ((), ())))
                dc = sq_s[:, pl.ds(c0, CW)] - (m2 + m2)
                lcg = c0 + lio
                dc = lax.cond(
                    c0 == (r0 // CW) * CW,
                    lambda d: jnp.where(lcg == r0 + rowio, d + 1e10, d),
                    lambda d: d, dc)
                dc = lax.cond(
                    c == NC - 1,
                    lambda d: jnp.where(lcg >= N, BIG, d),
                    lambda d: d, dc)

                def sel(k, cr):
                    dc, v8, i8, nv, ni = cr
                    mnp = jnp.minimum(
                        jnp.min(dc, axis=1, keepdims=True),
                        jnp.min(v8, axis=1, keepdims=True))
                    cd = jnp.min(jnp.where(dc == mnp, lcg, ISENT),
                                 axis=1, keepdims=True)
                    cv = jnp.min(jnp.where(v8 == mnp, i8, ISENT),
                                 axis=1, keepdims=True)
                    a = jnp.minimum(cd, cv)
                    dc = jnp.where(lcg == a, BIG, dc)
                    v8 = jnp.where(i8 == a, BIG, v8)
                    nv = jnp.where(k8 == k, mnp, nv)
                    ni = jnp.where(k8 == k, a, ni)
                    return dc, v8, i8, nv, ni

                nv0 = jnp.full((T, K), BIG, jnp.float32)
                ni0 = jnp.full((T, K), ISENT, jnp.int32)
                _, _, _, nv, ni = lax.fori_loop(0, K, sel,
                                                (dc, v8, i8, nv0, ni0))
                return nv, ni

            v80 = jnp.full((T, K), BIG, jnp.float32)
            i80 = jnp.full((T, K), ISENT, jnp.int32)
            _, i8 = lax.fori_loop(0, NC, chunk_body, (v80, i80))
            idx_s[pl.ds(r0, T), :] = i8
            agg = agg_pass(i8)
            h1t = jax.nn.relu(agg + hw_s[pl.ds(r0, T), :] + b1i)
            h1_s[pl.ds(r0, T), :] = h1t
            return 0

        lax.fori_loop(0, NT, p1_tile, 0)
        set_hw(jnp.dot(h1_s[:], W2i) * (1.0 / 9.0))

        def p2_tile(t, _):
            r0 = t * T
            idxt = idx_s[pl.ds(r0, T), :]
            agg = agg_pass(idxt)
            out = (agg + hw_s[pl.ds(r0, T), :] + b2i
                   + h_ref[0, pl.ds(r0, T), :])
            h_ref[0, pl.ds(r0, T), :] = out
            return 0

        lax.fori_loop(0, NT, p2_tile, 0)
        return 0

    lax.fori_loop(0, NBLK, block_body, 0)

    hf = h_ref[0]
    z = jnp.dot(hf, Wp1_ref[:]) + bp1_ref[:]
    mu = jnp.mean(z, axis=1, keepdims=True)
    zc = z - mu
    var = jnp.mean(zc * zc, axis=1, keepdims=True)
    z = zc / jnp.sqrt(var + 1e-5) * lng_ref[:] + lnb_ref[:]
    z = jax.nn.relu(z)
    g = jnp.dot(z, Wp2_ref[:]) + bp2_ref[:]
    off = jnp.tanh(g) * 0.1
    off_ref[0] = off[:, :8]


@jax.jit
def kernel(xyz, features, W1, b1, W2, b2, Wp1, bp1, ln_g, ln_b, Wp2, bp2):
    h0 = jnp.transpose(features, (0, 2, 1))
    h0 = jnp.pad(h0, ((0, 0), (0, NP - N), (0, 0)))
    Wp2p = jnp.pad(Wp2, ((0, 0), (0, 128 - Wp2.shape[1])))
    bp2p = jnp.pad(bp2, (0, 128 - bp2.shape[0]))[None, :]
    bp1r = bp1[None, :]
    lngr = ln_g[None, :]
    lnbr = ln_b[None, :]

    hout, off = pl.pallas_call(
        _net_kernel,
        grid=(B,),
        in_specs=[
            pl.BlockSpec((1, NP, DIM), lambda b: (b, 0, 0)),
            pl.BlockSpec((NBLK, DIM, DIM), lambda b: (0, 0, 0)),
            pl.BlockSpec((NBLK, DIM), lambda b: (0, 0)),
            pl.BlockSpec((NBLK, DIM, DIM), lambda b: (0, 0, 0)),
            pl.BlockSpec((NBLK, DIM), lambda b: (0, 0)),
            pl.BlockSpec((DIM, 128), lambda b: (0, 0)),
            pl.BlockSpec((1, 128), lambda b: (0, 0)),
            pl.BlockSpec((1, 128), lambda b: (0, 0)),
            pl.BlockSpec((1, 128), lambda b: (0, 0)),
            pl.BlockSpec((DIM, 128), lambda b: (0, 0)),
            pl.BlockSpec((1, 128), lambda b: (0, 0)),
        ],
        out_specs=[
            pl.BlockSpec((1, NP, DIM), lambda b: (b, 0, 0)),
            pl.BlockSpec((1, NP, 8), lambda b: (b, 0, 0)),
        ],
        out_shape=[
            jax.ShapeDtypeStruct((B, NP, DIM), jnp.float32),
            jax.ShapeDtypeStruct((B, NP, 8), jnp.float32),
        ],
        scratch_shapes=[
            pltpu.VMEM((NP, DIM), jnp.float32),
            pltpu.VMEM((NP, DIM), jnp.float32),
            pltpu.VMEM((NP, K), jnp.int32),
            pltpu.VMEM((1, NP), jnp.float32),
            pltpu.VMEM((NP, DIM), jnp.bfloat16),
            pltpu.VMEM((NP, DIM), jnp.bfloat16),
            pltpu.VMEM((NP, DIM), jnp.bfloat16),
        ],
        compiler_params=pltpu.CompilerParams(
            dimension_semantics=("parallel",)),
    )(h0, W1, b1, W2, b2, Wp1, bp1r, lngr, lnbr, Wp2p, bp2p)

    h = hout[:, :N, :]
    new_features = jnp.transpose(h, (0, 2, 1))
    off6 = off[:, :N, :6].reshape(B, N, 3, 2)
    new_xyz = (xyz[:, :, None, :]
               + jnp.transpose(off6, (0, 2, 3, 1))).reshape(B, 3, 2 * N)
    return (new_xyz, new_features)


# speedup vs baseline: 1.2827x; 1.0243x over previous
import jax
import jax.numpy as jnp
from jax import lax
from jax.experimental import pallas as pl
from jax.experimental.pallas import tpu as pltpu

B, N, K, DIM, NBLK = 4, 2500, 8, 128, 4
NP = 2560
T = 640
NT = NP // T
CW = 512
NC = NP // CW
BIG = 1e30
ISENT = NP


def _net_kernel(h0_ref, W1_ref, b1_ref, W2_ref, b2_ref,
                Wp1_ref, bp1_ref, lng_ref, lnb_ref, Wp2_ref, bp2_ref,
                h_ref, off_ref, h1_s, hw_s, idx_s, sq_s, hwa_s, hwb_s, hwc_s):
    h_ref[0] = h0_ref[0]

    lio = lax.broadcasted_iota(jnp.int32, (T, CW), 1)
    rowio = lax.broadcasted_iota(jnp.int32, (T, 1), 0)
    k8 = lax.broadcasted_iota(jnp.int32, (1, K), 1)

    def set_hw(hw):
        hw_s[:] = hw
        a = hw.astype(jnp.bfloat16)
        r = hw - a.astype(jnp.float32)
        b = r.astype(jnp.bfloat16)
        c = (r - b.astype(jnp.float32)).astype(jnp.bfloat16)
        hwa_s[:] = a
        hwb_s[:] = b
        hwc_s[:] = c

    def agg_pass(i8):
        def agg_chunk(c, agg):
            c0 = c * CW
            acc = jnp.zeros((T, CW), jnp.bfloat16)
            lcg = c0 + lio
            for k in range(K):
                acc = acc + (lcg == i8[:, k:k + 1]).astype(jnp.bfloat16)
            p = jnp.dot(acc, hwa_s[pl.ds(c0, CW), :],
                        preferred_element_type=jnp.float32)
            p = p + jnp.dot(acc, hwb_s[pl.ds(c0, CW), :],
                            preferred_element_type=jnp.float32)
            p = p + jnp.dot(acc, hwc_s[pl.ds(c0, CW), :],
                            preferred_element_type=jnp.float32)
            return agg + p

        return lax.fori_loop(0, NC, agg_chunk,
                             jnp.zeros((T, DIM), jnp.float32))

    def block_body(blk, _):
        W1i = W1_ref[blk]
        W2i = W2_ref[blk]
        b1i = b1_ref[pl.ds(blk, 1), :]
        b2i = b2_ref[pl.ds(blk, 1), :]
        hb = h_ref[0]
        sq_s[:] = jnp.sum(hb * hb, axis=1)[None, :]
        set_hw(jnp.dot(hb, W1i) * (1.0 / 9.0))

        def p1_tile(t, _):
            r0 = t * T
            ht = h_ref[0, pl.ds(r0, T), :]

            def chunk_body(c, carry):
                v8, i8 = carry
                c0 = c * CW
                hbc = h_ref[0, pl.ds(c0, CW), :]
                m2 = lax.dot_general(ht, hbc, (((1,), (1,)), ((), ())))
                dc = sq_s[:, pl.ds(c0, CW)] - (m2 + m2)
                lcg = c0 + lio
                dc = lax.cond(
                    jnp.logical_and(c0 + CW > r0, c0 < r0 + T),
                    lambda d: jnp.where(lcg == r0 + rowio, d + 1e10, d),
                    lambda d: d, dc)
                dc = lax.cond(
                    c == NC - 1,
                    lambda d: jnp.where(lcg >= N, BIG, d),
                    lambda d: d, dc)

                def sel(k, cr):
                    dc, v8, i8, nv, ni = cr
                    mnp = jnp.minimum(
                        jnp.min(dc, axis=1, keepdims=True),
                        jnp.min(v8, axis=1, keepdims=True))
                    cd = jnp.min(jnp.where(dc == mnp, lcg, ISENT),
                                 axis=1, keepdims=True)
                    cv = jnp.min(jnp.where(v8 == mnp, i8, ISENT),
                                 axis=1, keepdims=True)
                    a = jnp.minimum(cd, cv)
                    dc = jnp.where(lcg == a, BIG, dc)
                    v8 = jnp.where(i8 == a, BIG, v8)
                    nv = jnp.where(k8 == k, mnp, nv)
                    ni = jnp.where(k8 == k, a, ni)
                    return dc, v8, i8, nv, ni

                nv0 = jnp.full((T, K), BIG, jnp.float32)
                ni0 = jnp.full((T, K), ISENT, jnp.int32)
                _, _, _, nv, ni = lax.fori_loop(0, K, sel,
                                                (dc, v8, i8, nv0, ni0))
                return nv, ni

            v80 = jnp.full((T, K), BIG, jnp.float32)
            i80 = jnp.full((T, K), ISENT, jnp.int32)
            _, i8 = lax.fori_loop(0, NC, chunk_body, (v80, i80))
            idx_s[pl.ds(r0, T), :] = i8
            agg = agg_pass(i8)
            h1t = jax.nn.relu(agg + hw_s[pl.ds(r0, T), :] + b1i)
            h1_s[pl.ds(r0, T), :] = h1t
            return 0

        lax.fori_loop(0, NT, p1_tile, 0)
        set_hw(jnp.dot(h1_s[:], W2i) * (1.0 / 9.0))

        def p2_tile(t, _):
            r0 = t * T
            idxt = idx_s[pl.ds(r0, T), :]
            agg = agg_pass(idxt)
            out = (agg + hw_s[pl.ds(r0, T), :] + b2i
                   + h_ref[0, pl.ds(r0, T), :])
            h_ref[0, pl.ds(r0, T), :] = out
            return 0

        lax.fori_loop(0, NT, p2_tile, 0)
        return 0

    lax.fori_loop(0, NBLK, block_body, 0)

    hf = h_ref[0]
    z = jnp.dot(hf, Wp1_ref[:]) + bp1_ref[:]
    mu = jnp.mean(z, axis=1, keepdims=True)
    zc = z - mu
    var = jnp.mean(zc * zc, axis=1, keepdims=True)
    z = zc / jnp.sqrt(var + 1e-5) * lng_ref[:] + lnb_ref[:]
    z = jax.nn.relu(z)
    g = jnp.dot(z, Wp2_ref[:]) + bp2_ref[:]
    off = jnp.tanh(g) * 0.1
    off_ref[0] = off[:, :8]


@jax.jit
def kernel(xyz, features, W1, b1, W2, b2, Wp1, bp1, ln_g, ln_b, Wp2, bp2):
    h0 = jnp.transpose(features, (0, 2, 1))
    h0 = jnp.pad(h0, ((0, 0), (0, NP - N), (0, 0)))
    Wp2p = jnp.pad(Wp2, ((0, 0), (0, 128 - Wp2.shape[1])))
    bp2p = jnp.pad(bp2, (0, 128 - bp2.shape[0]))[None, :]
    bp1r = bp1[None, :]
    lngr = ln_g[None, :]
    lnbr = ln_b[None, :]

    hout, off = pl.pallas_call(
        _net_kernel,
        grid=(B,),
        in_specs=[
            pl.BlockSpec((1, NP, DIM), lambda b: (b, 0, 0)),
            pl.BlockSpec((NBLK, DIM, DIM), lambda b: (0, 0, 0)),
            pl.BlockSpec((NBLK, DIM), lambda b: (0, 0)),
            pl.BlockSpec((NBLK, DIM, DIM), lambda b: (0, 0, 0)),
            pl.BlockSpec((NBLK, DIM), lambda b: (0, 0)),
            pl.BlockSpec((DIM, 128), lambda b: (0, 0)),
            pl.BlockSpec((1, 128), lambda b: (0, 0)),
            pl.BlockSpec((1, 128), lambda b: (0, 0)),
            pl.BlockSpec((1, 128), lambda b: (0, 0)),
            pl.BlockSpec((DIM, 128), lambda b: (0, 0)),
            pl.BlockSpec((1, 128), lambda b: (0, 0)),
        ],
        out_specs=[
            pl.BlockSpec((1, NP, DIM), lambda b: (b, 0, 0)),
            pl.BlockSpec((1, NP, 8), lambda b: (b, 0, 0)),
        ],
        out_shape=[
            jax.ShapeDtypeStruct((B, NP, DIM), jnp.float32),
            jax.ShapeDtypeStruct((B, NP, 8), jnp.float32),
        ],
        scratch_shapes=[
            pltpu.VMEM((NP, DIM), jnp.float32),
            pltpu.VMEM((NP, DIM), jnp.float32),
            pltpu.VMEM((NP, K), jnp.int32),
            pltpu.VMEM((1, NP), jnp.float32),
            pltpu.VMEM((NP, DIM), jnp.bfloat16),
            pltpu.VMEM((NP, DIM), jnp.bfloat16),
            pltpu.VMEM((NP, DIM), jnp.bfloat16),
        ],
        compiler_params=pltpu.CompilerParams(
            dimension_semantics=("parallel",)),
    )(h0, W1, b1, W2, b2, Wp1, bp1r, lngr, lnbr, Wp2p, bp2p)

    h = hout[:, :N, :]
    new_features = jnp.transpose(h, (0, 2, 1))
    off6 = off[:, :N, :6].reshape(B, N, 3, 2)
    new_xyz = (xyz[:, :, None, :]
               + jnp.transpose(off6, (0, 2, 3, 1))).reshape(B, 3, 2 * N)
    return (new_xyz, new_features)
